# baseline (speedup 1.0000x reference)
import functools

import jax
import jax.numpy as jnp
from jax import lax
from jax.experimental import pallas as pl
from jax.experimental.pallas import tpu as pltpu
from jax.experimental.pallas import tpu_sc as plsc

jax.config.update("jax_enable_x64", True)

L_SEQ = 32768
NS = 16
CHUNK = L_SEQ // NS
LANES = 16
N_VECS = CHUNK // LANES
BYTE_BASE = 10


def _sc_body(tokens_hbm, marker_hbm, best_hbm, out_hbm,
             chunk_v, marker_v, best_v, stage_v, win_v, out_v):
    sid = lax.axis_index("s")
    base = sid * CHUNK
    pltpu.sync_copy(tokens_hbm.at[pl.ds(base, CHUNK)], chunk_v)
    pltpu.sync_copy(marker_hbm, marker_v)
    m = marker_v[...]
    lane = lax.broadcasted_iota(jnp.int32, (LANES,), 0)

    def scan_step(_, carry):
        best, off = carry
        v = chunk_v[pl.ds(off, LANES)]
        gidx = base + off + lane
        best = jnp.maximum(best, jnp.where(v == m, gidx, jnp.int32(-1)))
        return best, off + jnp.int32(LANES)

    best, _ = lax.fori_loop(
        0, N_VECS, scan_step,
        (jnp.full((LANES,), -1, jnp.int32), jnp.int32(0)))
    best_v[...] = best
    pltpu.sync_copy(best_v, best_hbm.at[pl.ds(sid * LANES, LANES)])
    plsc.subcore_barrier()

    @pl.when(sid == 0)
    def _():
        pltpu.sync_copy(best_hbm, stage_v)
        red = stage_v[pl.ds(0, LANES)]
        for i in range(1, NS):
            red = jnp.maximum(red, stage_v[pl.ds(i * LANES, LANES)])
        pos = jnp.max(red)
        found = pos >= 0
        pos0 = jnp.maximum(pos, 0)
        start = pl.multiple_of(
            jnp.minimum((pos0 + 1) & ~7, L_SEQ - LANES), 8)
        pltpu.sync_copy(tokens_hbm.at[pl.ds(start, LANES)], win_v)
        k = jnp.minimum(lane, 3)
        local = jnp.clip(pos0 + 1 + k, 0, L_SEQ - 1) - start
        toks = plsc.load_gather(win_v, [local])
        byte_vals = jnp.clip(toks - jnp.int32(BYTE_BASE), 0, 255)
        one = jnp.int32(1)
        zero = jnp.int32(0)
        c256 = jnp.int32(256)
        mult_lo = (jnp.where(lane == 0, one, zero)
                   + jnp.where(lane == 1, c256, zero))
        mult_hi = (jnp.where(lane == 2, one, zero)
                   + jnp.where(lane == 3, c256, zero))
        low16 = jnp.sum(byte_vals * mult_lo, dtype=jnp.int32)
        high16 = jnp.sum(byte_vals * mult_hi, dtype=jnp.int32)
        found_i32 = jnp.where(found, one, zero)
        out_v[...] = (jnp.where(lane == 0, low16, zero)
                      + jnp.where(lane == 1, high16, zero)
                      + jnp.where(lane == 2, found_i32, zero))
        pltpu.sync_copy(out_v, out_hbm)


@functools.partial(
    pl.kernel,
    out_type=(jax.ShapeDtypeStruct((NS * LANES,), jnp.int32),
              jax.ShapeDtypeStruct((LANES,), jnp.int32)),
    mesh=plsc.VectorSubcoreMesh(core_axis_name="c", subcore_axis_name="s",
                                num_cores=1, num_subcores=NS),
    scratch_types=[
        pltpu.VMEM((CHUNK,), jnp.int32),
        pltpu.VMEM((LANES,), jnp.int32),
        pltpu.VMEM((LANES,), jnp.int32),
        pltpu.VMEM((NS * LANES,), jnp.int32),
        pltpu.VMEM((LANES,), jnp.int32),
        pltpu.VMEM((LANES,), jnp.int32),
    ],
    compiler_params=pltpu.CompilerParams(needs_layout_passes=False),
)
def _reader_kernel(tokens_hbm, marker_hbm, best_hbm, out_hbm, *scratch):
    _sc_body(tokens_hbm, marker_hbm, best_hbm, out_hbm, *scratch)


def kernel(context_tokens, marker):
    tokens = context_tokens[0].astype(jnp.int32)
    marker_arr = jnp.full((LANES,), marker, dtype=jnp.int32)
    _, out = _reader_kernel(tokens, marker_arr)
    val = out[0].astype(jnp.int64) + (out[1].astype(jnp.int64) << 16)
    return jnp.where(out[2] > 0, val, jnp.int64(0))
